# baseline (speedup 1.0000x reference)
import functools

import jax
import jax.numpy as jnp
from jax import lax
from jax.experimental import pallas as pl
from jax.experimental.pallas import tpu as pltpu
from jax.experimental.pallas import tpu_sc as plsc

B, L, D = 4096, 200, 32
SEQ = L + 2
V_DIS, V_CNT, V_AGE, V_SEX = 100000, 50, 100, 3
NC, NS = 2, 16
NW = NC * NS
CHUNK = 128
N_CHUNKS = B * SEQ // CHUNK
PER_W = N_CHUNKS // NW
SIDE_ROWS = 160
LANES = 16
GROUPS = CHUNK // LANES

_mesh = plsc.VectorSubcoreMesh(
    core_axis_name="c", subcore_axis_name="s", num_cores=NC, num_subcores=NS
)


@functools.partial(
    pl.kernel,
    mesh=_mesh,
    compiler_params=pltpu.CompilerParams(
        needs_layout_passes=False, use_tc_tiling_on_sc=False
    ),
    out_type=jax.ShapeDtypeStruct((N_CHUNKS, CHUNK * D), jnp.float32),
    scratch_types=[
        pltpu.VMEM((SIDE_ROWS * D,), jnp.float32),
        pltpu.VMEM((CHUNK * D,), jnp.float32),
        pltpu.VMEM((CHUNK * D,), jnp.float32),
        pltpu.VMEM((CHUNK,), jnp.int32),
        pltpu.VMEM((CHUNK,), jnp.int32),
        pltpu.VMEM((CHUNK,), jnp.int32),
        pltpu.VMEM((CHUNK,), jnp.int32),
        pltpu.VMEM((CHUNK, D), jnp.float32),
        pltpu.VMEM((CHUNK, D), jnp.float32),
        pltpu.SemaphoreType.DMA,
        pltpu.SemaphoreType.DMA,
        pltpu.SemaphoreType.DMA,
        pltpu.SemaphoreType.DMA,
        pltpu.SemaphoreType.DMA,
        pltpu.SemaphoreType.DMA,
    ],
)
def _sc_embed(wdis_hbm, side_hbm, idxm_hbm, idxs_hbm, out_hbm,
              side_v, fl0, fl1, im0, im1, is0, is1, b0, b1,
              si0, si1, sg0, sg1, so0, so1):
    wid = lax.axis_index("s") * NC + lax.axis_index("c")
    base = wid * PER_W
    im, isv, buf, fl = [im0, im1], [is0, is1], [b0, b1], [fl0, fl1]
    sidx, sg, so = [si0, si1], [sg0, sg1], [so0, so1]

    pltpu.sync_copy(side_hbm, side_v)

    def idx_start(j, s):
        pltpu.async_copy(idxm_hbm.at[base + j], im[s], sidx[s])
        pltpu.async_copy(idxs_hbm.at[base + j], isv[s], sidx[s])

    def idx_wait(s):
        pltpu.make_async_copy(idxm_hbm.at[0], im[s], sidx[s]).wait()
        pltpu.make_async_copy(idxs_hbm.at[0], isv[s], sidx[s]).wait()

    def gather_start(s):
        pltpu.async_copy(wdis_hbm.at[im[s]], buf[s], sg[s])

    def gather_wait(s):
        pltpu.make_async_copy(wdis_hbm.at[im[s]], buf[s], sg[s]).wait()

    def out_start(j, s):
        pltpu.async_copy(fl[s], out_hbm.at[base + j], so[s])

    def out_wait(s):
        pltpu.make_async_copy(fl[s], out_hbm.at[0], so[s]).wait()

    iota32 = lax.iota(jnp.int32, LANES) * D

    def compute(s):
        @plsc.parallel_loop(0, CHUNK, step=1, unroll=8)
        def _(r):
            for h in range(D // LANES):
                fl[s][pl.ds(r * D + h * LANES, LANES)] = buf[s][
                    r, pl.ds(h * LANES, LANES)
                ]

        for g in range(GROUPS):
            cb = isv[s][pl.ds(g * LANES, LANES)]
            rb = jnp.full((LANES,), g * LANES * D, jnp.int32) + iota32
            for d in range(D):
                vals = plsc.load_gather(side_v, [cb + d])
                plsc.addupdate_scatter(fl[s], [rb + d], vals)

    idx_start(0, 0)
    idx_start(1, 1)
    idx_wait(0)
    gather_start(0)

    def pair_body(p, carry):
        for s in (0, 1):
            j = 2 * p + s
            o = 1 - s

            @pl.when(j + 1 < PER_W)
            def _():
                idx_wait(o)

                @pl.when(j >= 1)
                def _():
                    out_wait(o)

                gather_start(o)

            gather_wait(s)
            compute(s)
            out_start(j, s)

            @pl.when(j + 2 < PER_W)
            def _():
                idx_start(j + 2, s)

        return carry

    lax.fori_loop(0, PER_W // 2, pair_body, 0)
    out_wait(0)
    out_wait(1)


def kernel(diseases, counts, age, sex, W_diseases, W_counts, W_age, W_sex):
    idx_main = jnp.concatenate(
        [diseases, jnp.zeros((B, 2), jnp.int32)], axis=1
    ).reshape(N_CHUNKS, CHUNK)
    idx_side = (
        jnp.concatenate(
            [counts, V_CNT + age[:, None], V_CNT + V_AGE + sex[:, None]], axis=1
        )
        * D
    ).reshape(N_CHUNKS, CHUNK)
    wbase = W_diseases[0]
    side = jnp.concatenate(
        [
            W_counts,
            W_age - wbase,
            W_sex - wbase,
            jnp.zeros((SIDE_ROWS - V_CNT - V_AGE - V_SEX, D), jnp.float32),
        ],
        axis=0,
    ).reshape(-1)
    out = _sc_embed(W_diseases, side, idx_main, idx_side)
    return out.reshape(B, SEQ, D)


# speedup vs baseline: 1.3710x; 1.3710x over previous
import functools

import jax
import jax.numpy as jnp
import numpy as np
from jax import lax
from jax.experimental import pallas as pl
from jax.experimental.pallas import tpu as pltpu
from jax.experimental.pallas import tpu_sc as plsc

B, L, D = 4096, 200, 32
SEQ = L + 2
V_DIS, V_CNT, V_AGE, V_SEX = 100000, 50, 100, 3
NC, NS = 2, 16
NW = NC * NS
CHUNK = 128
N_CHUNKS = B * SEQ // CHUNK
PER_W = N_CHUNKS // NW
SIDE_ROWS = 160
LANES = 16

_PERM = np.empty((D,), np.int32)
_PERM[0::2] = np.arange(16)
_PERM[1::2] = 16 + np.arange(16)

_mesh = plsc.VectorSubcoreMesh(
    core_axis_name="c", subcore_axis_name="s", num_cores=NC, num_subcores=NS
)


@functools.partial(
    pl.kernel,
    mesh=_mesh,
    compiler_params=pltpu.CompilerParams(
        needs_layout_passes=False, use_tc_tiling_on_sc=False
    ),
    out_type=jax.ShapeDtypeStruct((N_CHUNKS, CHUNK * D), jnp.float32),
    scratch_types=[
        pltpu.VMEM((CHUNK * D,), jnp.float32),
        pltpu.VMEM((CHUNK * D,), jnp.float32),
        pltpu.VMEM((CHUNK,), jnp.int32),
        pltpu.VMEM((CHUNK,), jnp.int32),
        pltpu.VMEM((CHUNK,), jnp.int32),
        pltpu.VMEM((CHUNK,), jnp.int32),
        pltpu.VMEM((CHUNK, D), jnp.bfloat16),
        pltpu.VMEM((CHUNK, D), jnp.bfloat16),
        pltpu.VMEM((CHUNK, D), jnp.bfloat16),
        pltpu.VMEM((CHUNK, D), jnp.bfloat16),
        pltpu.SemaphoreType.DMA,
        pltpu.SemaphoreType.DMA,
        pltpu.SemaphoreType.DMA,
        pltpu.SemaphoreType.DMA,
        pltpu.SemaphoreType.DMA,
        pltpu.SemaphoreType.DMA,
    ],
)
def _sc_embed(wdis_hbm, side_hbm, idxm_hbm, idxs_hbm, out_hbm,
              fl0, fl1, im0, im1, is0, is1, b0, b1, c0, c1,
              si0, si1, sg0, sg1, so0, so1):
    wid = lax.axis_index("s") * NC + lax.axis_index("c")
    base = wid * PER_W
    im, isv, buf, buf2 = [im0, im1], [is0, is1], [b0, b1], [c0, c1]
    fl = [fl0, fl1]
    sidx, sg, so = [si0, si1], [sg0, sg1], [so0, so1]

    def idx_start(j, s):
        pltpu.async_copy(idxm_hbm.at[base + j], im[s], sidx[s])
        pltpu.async_copy(idxs_hbm.at[base + j], isv[s], sidx[s])

    def idx_wait(s):
        pltpu.make_async_copy(idxm_hbm.at[0], im[s], sidx[s]).wait()
        pltpu.make_async_copy(idxs_hbm.at[0], isv[s], sidx[s]).wait()

    def gather_start(s):
        pltpu.async_copy(wdis_hbm.at[im[s]], buf[s], sg[s])
        pltpu.async_copy(side_hbm.at[isv[s]], buf2[s], sg[s])

    def gather_wait(s):
        pltpu.make_async_copy(wdis_hbm.at[im[s]], buf[s], sg[s]).wait()
        pltpu.make_async_copy(side_hbm.at[isv[s]], buf2[s], sg[s]).wait()

    def out_start(j, s):
        pltpu.async_copy(fl[s], out_hbm.at[base + j], so[s])

    def out_wait(s):
        pltpu.make_async_copy(fl[s], out_hbm.at[0], so[s]).wait()

    def compute(s):
        @plsc.parallel_loop(0, CHUNK, step=1, unroll=4)
        def _(r):
            a, b = plsc.unpack(buf[s][r, :], format=plsc.PackFormat.INTERLEAVED)
            sa, sb = plsc.unpack(
                buf2[s][r, :], format=plsc.PackFormat.INTERLEAVED
            )
            fl[s][pl.ds(r * D, LANES)] = a + sa
            fl[s][pl.ds(r * D + LANES, LANES)] = b + sb

    idx_start(0, 0)
    idx_start(1, 1)
    idx_wait(0)
    gather_start(0)

    def pair_body(p, carry):
        for s in (0, 1):
            j = 2 * p + s
            o = 1 - s

            @pl.when(j + 1 < PER_W)
            def _():
                idx_wait(o)

                @pl.when(j >= 1)
                def _():
                    out_wait(o)

                gather_start(o)

            gather_wait(s)
            compute(s)
            out_start(j, s)

            @pl.when(j + 2 < PER_W)
            def _():
                idx_start(j + 2, s)

        return carry

    lax.fori_loop(0, PER_W // 2, pair_body, 0)
    out_wait(0)
    out_wait(1)


def kernel(diseases, counts, age, sex, W_diseases, W_counts, W_age, W_sex):
    idx_main = jnp.concatenate(
        [diseases, jnp.zeros((B, 2), jnp.int32)], axis=1
    ).reshape(N_CHUNKS, CHUNK)
    idx_side = jnp.concatenate(
        [counts, V_CNT + age[:, None], V_CNT + V_AGE + sex[:, None]], axis=1
    ).reshape(N_CHUNKS, CHUNK)
    wbase = W_diseases[0]
    side = jnp.concatenate(
        [
            W_counts,
            W_age - wbase,
            W_sex - wbase,
            jnp.zeros((SIDE_ROWS - V_CNT - V_AGE - V_SEX, D), jnp.float32),
        ],
        axis=0,
    )
    perm = jnp.asarray(_PERM)
    wdis_bf = W_diseases[:, perm].astype(jnp.bfloat16)
    side_bf = side[:, perm].astype(jnp.bfloat16)
    out = _sc_embed(wdis_bf, side_bf, idx_main, idx_side)
    return out.reshape(B, SEQ, D)
